# Initial kernel scaffold; baseline (speedup 1.0000x reference)
import jax
import jax.numpy as jnp
from jax.experimental import pallas as pl


def kernel(x, edge_index, W_lin, b_lin, W_ih, W_hh, b_ih, b_hh, W_c, b_c):
    raise NotImplementedError("write your pallas kernel here")


# speedup vs baseline: 21.2169x; 21.2169x over previous
import functools

import jax
import jax.numpy as jnp
from jax import lax
from jax.experimental import pallas as pl
from jax.experimental.pallas import tpu as pltpu
from jax.experimental.pallas import tpu_sc as plsc

N = 50000
E = 1600000
H = 32
NCLS = 10
T = 5

NC = 2
NS = 16
NW = NC * NS

N_P = 50176
ROWS_PER_TILE = N_P // NS

K = 4
CHUNK_E = K * 128
CHUNKS = -(-E // (NW * CHUNK_E))
E_PAD = NW * CHUNKS * CHUNK_E
IDX_ROWS = E_PAD // 128
ROWS_PER_W = IDX_ROWS // NW

BN = 3136
GRID = N_P // BN

_mesh = plsc.VectorSubcoreMesh(
    core_axis_name="c", subcore_axis_name="s", num_cores=NC, num_subcores=NS
)


@functools.partial(
    pl.kernel,
    out_type=jax.ShapeDtypeStruct((NC, N_P, H), jnp.float32),
    mesh=_mesh,
    compiler_params=pltpu.CompilerParams(use_tc_tiling_on_sc=False),
    scratch_types=[
        pltpu.VMEM_SHARED((N_P, H), jnp.float32),
        pltpu.VMEM((K, 128), jnp.int32),
        pltpu.VMEM((K, 128), jnp.int32),
        pltpu.VMEM((CHUNK_E, H), jnp.float32),
        pltpu.SemaphoreType.DMA,
    ],
)
def _edge_agg(a_hbm, src_hbm, dst_hbm, z_hbm, out_hbm, acc, src_v, dst_v, rows_v, sem):
    cid = lax.axis_index("c")
    sid = lax.axis_index("s")
    wid = cid * NS + sid
    tile_r0 = sid * ROWS_PER_TILE

    pltpu.sync_copy(z_hbm, acc.at[pl.ds(tile_r0, ROWS_PER_TILE)])
    plsc.subcore_barrier()

    base = wid * ROWS_PER_W

    def chunk_body(ci, carry):
        r0 = base + ci * K
        pltpu.sync_copy(src_hbm.at[pl.ds(r0, K)], src_v)
        pltpu.sync_copy(dst_hbm.at[pl.ds(r0, K)], dst_v)
        cps = [
            pltpu.async_copy(
                a_hbm.at[src_v.at[j]], rows_v.at[pl.ds(j * 128, 128)], sem
            )
            for j in range(K)
        ]
        for cp in cps:
            cp.wait()
        for j in range(K):
            pltpu.sync_copy(
                rows_v.at[pl.ds(j * 128, 128)], acc.at[dst_v.at[j]], add=True
            )
        return carry

    lax.fori_loop(0, CHUNKS, chunk_body, 0)
    plsc.subcore_barrier()

    pltpu.sync_copy(
        acc.at[pl.ds(tile_r0, ROWS_PER_TILE)],
        out_hbm.at[cid, pl.ds(tile_r0, ROWS_PER_TILE)],
    )


def _dot(a, b):
    return lax.dot_general(
        a,
        b,
        (((1,), (0,)), ((), ())),
        precision=lax.Precision.DEFAULT,
        preferred_element_type=jnp.float32,
    )


def _init_body(x_ref, wcol_ref, bl_ref, h0_ref, a0_ref):
    xc = x_ref[...]
    col = lax.broadcasted_iota(jnp.int32, (BN, H), 1)
    h0_ref[...] = jnp.where(col == 0, xc, 0.0)
    a0_ref[...] = xc * wcol_ref[...] + bl_ref[...]


def _step_body(
    h_ref, a_ref, p0_ref, p1_ref, wih_ref, whh_ref, wl_ref, bih_ref, bhh_ref, bl_ref,
    hn_ref, an_ref,
):
    h = h_ref[...]
    agg = p0_ref[...] + p1_ref[...] + a_ref[...]
    gi = _dot(agg, wih_ref[...]) + bih_ref[...]
    gh = _dot(h, whh_ref[...]) + bhh_ref[...]
    r = jax.nn.sigmoid(gi[:, :H] + gh[:, :H])
    z = jax.nn.sigmoid(gi[:, H : 2 * H] + gh[:, H : 2 * H])
    n = jnp.tanh(gi[:, 2 * H :] + r * gh[:, 2 * H :])
    hn = (1.0 - z) * n + z * h
    hn_ref[...] = hn
    an_ref[...] = _dot(hn, wl_ref[...]) + bl_ref[...]


def _final_body(h_ref, wc_ref, bc_ref, o_ref):
    h = h_ref[...]
    row = lax.broadcasted_iota(jnp.int32, (N_P, H), 0)
    hr = jnp.where(row < N, jnp.maximum(h, 0.0), 0.0)
    s = jnp.sum(hr, axis=0, keepdims=True) * (1.0 / N)
    o_ref[...] = _dot(s, wc_ref[...]) + bc_ref[...]


_blk = lambda shape, imap: pl.BlockSpec(shape, imap)
_row_map = lambda i: (i, 0)
_full_map = lambda i: (0, 0)

_tc_init = pl.pallas_call(
    _init_body,
    grid=(GRID,),
    in_specs=[
        _blk((BN, 1), _row_map),
        _blk((1, H), _full_map),
        _blk((1, H), _full_map),
    ],
    out_specs=[_blk((BN, H), _row_map), _blk((BN, H), _row_map)],
    out_shape=[
        jax.ShapeDtypeStruct((N_P, H), jnp.float32),
        jax.ShapeDtypeStruct((N_P, H), jnp.float32),
    ],
)

_tc_step = pl.pallas_call(
    _step_body,
    grid=(GRID,),
    in_specs=[
        _blk((BN, H), _row_map),
        _blk((BN, H), _row_map),
        _blk((BN, H), _row_map),
        _blk((BN, H), _row_map),
        _blk((H, 3 * H), _full_map),
        _blk((H, 3 * H), _full_map),
        _blk((H, H), _full_map),
        _blk((1, 3 * H), _full_map),
        _blk((1, 3 * H), _full_map),
        _blk((1, H), _full_map),
    ],
    out_specs=[_blk((BN, H), _row_map), _blk((BN, H), _row_map)],
    out_shape=[
        jax.ShapeDtypeStruct((N_P, H), jnp.float32),
        jax.ShapeDtypeStruct((N_P, H), jnp.float32),
    ],
)

_tc_final = pl.pallas_call(
    _final_body,
    in_specs=[
        pl.BlockSpec((N_P, H), lambda: (0, 0)),
        pl.BlockSpec((H, NCLS), lambda: (0, 0)),
        pl.BlockSpec((1, NCLS), lambda: (0, 0)),
    ],
    out_specs=pl.BlockSpec((1, NCLS), lambda: (0, 0)),
    out_shape=jax.ShapeDtypeStruct((1, NCLS), jnp.float32),
)


def kernel(x, edge_index, W_lin, b_lin, W_ih, W_hh, b_ih, b_hh, W_c, b_c):
    xp = jnp.zeros((N_P, 1), jnp.float32).at[:N, 0].set(x)

    src = edge_index[0].astype(jnp.int32)
    dst = edge_index[1].astype(jnp.int32)
    npad = E_PAD - E
    pad_rows = N_P - N
    pad_idx = (N + (jnp.arange(npad, dtype=jnp.int32) % pad_rows)).astype(jnp.int32)
    srcp = jnp.concatenate([src, pad_idx]).reshape(IDX_ROWS, 128)
    dstp = jnp.concatenate([dst, pad_idx]).reshape(IDX_ROWS, 128)

    zrows = jnp.zeros((ROWS_PER_TILE, H), jnp.float32)
    wcol = W_lin[:, 0].reshape(1, H)
    bl = b_lin.reshape(1, H)
    WlT = W_lin.T
    WihT = W_ih.T
    WhhT = W_hh.T
    bih = b_ih.reshape(1, 3 * H)
    bhh = b_hh.reshape(1, 3 * H)
    WcT = W_c.T
    bc = b_c.reshape(1, NCLS)

    h, a = _tc_init(xp, wcol, bl)
    for _ in range(T):
        parts = _edge_agg(a, srcp, dstp, zrows)
        h, a = _tc_step(h, a, parts[0], parts[1], WihT, WhhT, WlT, bih, bhh, bl)
    return _tc_final(h, WcT, bc)


# speedup vs baseline: 34.1757x; 1.6108x over previous
import functools

import jax
import jax.numpy as jnp
from jax import lax
from jax.experimental import pallas as pl
from jax.experimental.pallas import tpu as pltpu
from jax.experimental.pallas import tpu_sc as plsc

N = 50000
E = 1600000
H = 32
NCLS = 10
T = 5

NC = 2
NS = 16
NW = NC * NS

N_P = 50176
ROWS_PER_TILE = N_P // NS

KR = 3
CE = KR * 128
NCHUNK = 132
ROWS_PER_W = NCHUNK * KR
E_PAD = NW * NCHUNK * CE
IDX_ROWS = E_PAD // 128

BN = 3136
GRID = N_P // BN

_mesh = plsc.VectorSubcoreMesh(
    core_axis_name="c", subcore_axis_name="s", num_cores=NC, num_subcores=NS
)


@functools.partial(
    pl.kernel,
    out_type=jax.ShapeDtypeStruct((NC, N_P, H), jnp.float32),
    mesh=_mesh,
    compiler_params=pltpu.CompilerParams(use_tc_tiling_on_sc=False),
    scratch_types=[
        pltpu.VMEM_SHARED((N_P, H), jnp.float32),
        [pltpu.VMEM((KR, 128), jnp.int32)] * 2,
        [pltpu.VMEM((KR, 128), jnp.int32)] * 2,
        [pltpu.VMEM((KR, 128), jnp.int32)] * 2,
        [pltpu.VMEM((KR, 128), jnp.int32)] * 2,
        [pltpu.VMEM((CE, H), jnp.float32)] * 2,
        [pltpu.SemaphoreType.DMA] * 2,
        [pltpu.SemaphoreType.DMA] * 2,
        [pltpu.SemaphoreType.DMA] * 2,
    ],
)
def _edge_agg(a_hbm, src_hbm, dst_hbm, z_hbm, out_hbm, acc, sstg, dstg, spriv,
              dpriv, rows, gsems, ssems, isems):
    cid = lax.axis_index("c")
    sid = lax.axis_index("s")
    wid = cid * NS + sid
    tile_r0 = sid * ROWS_PER_TILE

    pltpu.sync_copy(z_hbm, acc.at[pl.ds(tile_r0, ROWS_PER_TILE)])
    plsc.subcore_barrier()

    base = wid * ROWS_PER_W

    def idx_copies(c, h):
        r0 = base + c * KR
        return (pltpu.make_async_copy(src_hbm.at[pl.ds(r0, KR)], sstg[h],
                                      isems[h]),
                pltpu.make_async_copy(dst_hbm.at[pl.ds(r0, KR)], dstg[h],
                                      isems[h]))

    def vcopy(srcref, dstref):
        for r in range(KR):
            for k in range(8):
                dstref[r, pl.ds(k * 16, 16)] = srcref[r, pl.ds(k * 16, 16)]

    def gather_start(h):
        for r in range(KR):
            pltpu.async_copy(a_hbm.at[spriv[h].at[r]],
                             rows[h].at[pl.ds(r * 128, 128)], gsems[h])

    def gather_wait(h):
        pltpu.make_async_copy(z_hbm.at[pl.ds(0, CE)], rows[h], gsems[h]).wait()

    def scatter_start(h):
        for r in range(KR):
            pltpu.async_copy(rows[h].at[pl.ds(r * 128, 128)],
                             acc.at[dpriv[h].at[r]], ssems[h], add=True)

    def drain_scatter(h):
        pltpu.make_async_copy(z_hbm.at[pl.ds(0, CE)], rows[h], ssems[h]).wait()

    for h in (0, 1):
        for cp in idx_copies(h, h):
            cp.start()

    def pair_body(s, carry):
        for h in (0, 1):
            c = 2 * s + h

            @pl.when(s > 0)
            def _():
                drain_scatter(h)
            for cp in idx_copies(c, h):
                cp.wait()
            vcopy(sstg[h], spriv[h])
            vcopy(dstg[h], dpriv[h])
            gather_start(h)

            @pl.when(jnp.logical_or(s > 0, h > 0))
            def _():
                gather_wait(1 - h)
                scatter_start(1 - h)

            @pl.when(s < NCHUNK // 2 - 1)
            def _():
                for cp in idx_copies(c + 2, h):
                    cp.start()
        return carry

    lax.fori_loop(0, NCHUNK // 2, pair_body, 0)
    gather_wait(1)
    scatter_start(1)
    drain_scatter(0)
    drain_scatter(1)
    plsc.subcore_barrier()

    pltpu.sync_copy(
        acc.at[pl.ds(tile_r0, ROWS_PER_TILE)],
        out_hbm.at[cid, pl.ds(tile_r0, ROWS_PER_TILE)],
    )


def _dot(a, b):
    return lax.dot_general(
        a,
        b,
        (((1,), (0,)), ((), ())),
        precision=lax.Precision.DEFAULT,
        preferred_element_type=jnp.float32,
    )


def _init_body(x_ref, wcol_ref, bl_ref, h0_ref, a0_ref):
    xc = x_ref[...]
    col = lax.broadcasted_iota(jnp.int32, (BN, H), 1)
    h0_ref[...] = jnp.where(col == 0, xc, 0.0)
    a0_ref[...] = xc * wcol_ref[...] + bl_ref[...]


def _step_body(
    h_ref, a_ref, p0_ref, p1_ref, wih_ref, whh_ref, wl_ref, bih_ref, bhh_ref, bl_ref,
    hn_ref, an_ref,
):
    h = h_ref[...]
    agg = p0_ref[...] + p1_ref[...] + a_ref[...]
    gi = _dot(agg, wih_ref[...]) + bih_ref[...]
    gh = _dot(h, whh_ref[...]) + bhh_ref[...]
    r = jax.nn.sigmoid(gi[:, :H] + gh[:, :H])
    z = jax.nn.sigmoid(gi[:, H : 2 * H] + gh[:, H : 2 * H])
    n = jnp.tanh(gi[:, 2 * H :] + r * gh[:, 2 * H :])
    hn = (1.0 - z) * n + z * h
    hn_ref[...] = hn
    an_ref[...] = _dot(hn, wl_ref[...]) + bl_ref[...]


def _final_body(h_ref, wc_ref, bc_ref, o_ref):
    h = h_ref[...]
    row = lax.broadcasted_iota(jnp.int32, (N_P, H), 0)
    hr = jnp.where(row < N, jnp.maximum(h, 0.0), 0.0)
    s = jnp.sum(hr, axis=0, keepdims=True) * (1.0 / N)
    o_ref[...] = _dot(s, wc_ref[...]) + bc_ref[...]


_blk = lambda shape, imap: pl.BlockSpec(shape, imap)
_row_map = lambda i: (i, 0)
_full_map = lambda i: (0, 0)

_tc_init = pl.pallas_call(
    _init_body,
    grid=(GRID,),
    in_specs=[
        _blk((BN, 1), _row_map),
        _blk((1, H), _full_map),
        _blk((1, H), _full_map),
    ],
    out_specs=[_blk((BN, H), _row_map), _blk((BN, H), _row_map)],
    out_shape=[
        jax.ShapeDtypeStruct((N_P, H), jnp.float32),
        jax.ShapeDtypeStruct((N_P, H), jnp.float32),
    ],
)

_tc_step = pl.pallas_call(
    _step_body,
    grid=(GRID,),
    in_specs=[
        _blk((BN, H), _row_map),
        _blk((BN, H), _row_map),
        _blk((BN, H), _row_map),
        _blk((BN, H), _row_map),
        _blk((H, 3 * H), _full_map),
        _blk((H, 3 * H), _full_map),
        _blk((H, H), _full_map),
        _blk((1, 3 * H), _full_map),
        _blk((1, 3 * H), _full_map),
        _blk((1, H), _full_map),
    ],
    out_specs=[_blk((BN, H), _row_map), _blk((BN, H), _row_map)],
    out_shape=[
        jax.ShapeDtypeStruct((N_P, H), jnp.float32),
        jax.ShapeDtypeStruct((N_P, H), jnp.float32),
    ],
)

_tc_final = pl.pallas_call(
    _final_body,
    in_specs=[
        pl.BlockSpec((N_P, H), lambda: (0, 0)),
        pl.BlockSpec((H, NCLS), lambda: (0, 0)),
        pl.BlockSpec((1, NCLS), lambda: (0, 0)),
    ],
    out_specs=pl.BlockSpec((1, NCLS), lambda: (0, 0)),
    out_shape=jax.ShapeDtypeStruct((1, NCLS), jnp.float32),
)


def kernel(x, edge_index, W_lin, b_lin, W_ih, W_hh, b_ih, b_hh, W_c, b_c):
    xp = jnp.zeros((N_P, 1), jnp.float32).at[:N, 0].set(x)

    src = edge_index[0].astype(jnp.int32)
    dst = edge_index[1].astype(jnp.int32)
    npad = E_PAD - E
    pad_rows = N_P - N
    pad_idx = (N + (jnp.arange(npad, dtype=jnp.int32) % pad_rows)).astype(jnp.int32)
    srcp = jnp.concatenate([src, pad_idx]).reshape(IDX_ROWS, 128)
    dstp = jnp.concatenate([dst, pad_idx]).reshape(IDX_ROWS, 128)

    zrows = jnp.zeros((ROWS_PER_TILE, H), jnp.float32)
    wcol = W_lin[:, 0].reshape(1, H)
    bl = b_lin.reshape(1, H)
    WlT = W_lin.T
    WihT = W_ih.T
    WhhT = W_hh.T
    bih = b_ih.reshape(1, 3 * H)
    bhh = b_hh.reshape(1, 3 * H)
    WcT = W_c.T
    bc = b_c.reshape(1, NCLS)

    h, a = _tc_init(xp, wcol, bl)
    for _ in range(T):
        parts = _edge_agg(a, srcp, dstp, zrows)
        h, a = _tc_step(h, a, parts[0], parts[1], WihT, WhhT, WlT, bih, bhh, bl)
    return _tc_final(h, WcT, bc)
